# Initial kernel scaffold; baseline (speedup 1.0000x reference)
import jax
import jax.numpy as jnp
from jax.experimental import pallas as pl


def kernel(input_tokens, embedding):
    raise NotImplementedError("write your pallas kernel here")


# speedup vs baseline: 1.6841x; 1.6841x over previous
import functools

import jax
import jax.numpy as jnp
from jax import lax
from jax.experimental import pallas as pl
from jax.experimental.pallas import tpu as pltpu
from jax.experimental.pallas import tpu_sc as plsc

_NC = 2
_NS = 16
_NW = _NC * _NS
_CHUNK = 128


@functools.lru_cache(maxsize=None)
def _make_gather(B, D):
    assert B % (_NW * _CHUNK) == 0
    bpw = B // _NW
    steps = bpw // _CHUNK
    mesh = plsc.VectorSubcoreMesh(core_axis_name="c", subcore_axis_name="s")

    @functools.partial(
        pl.kernel,
        out_type=jax.ShapeDtypeStruct((B, D), jnp.float32),
        mesh=mesh,
        compiler_params=pltpu.CompilerParams(use_tc_tiling_on_sc=False),
        scratch_types=[
            pltpu.VMEM((steps, _CHUNK), jnp.int32),
            pltpu.VMEM((_CHUNK, D), jnp.float32),
            pltpu.SemaphoreType.DMA,
        ],
    )
    def gather_kernel(idx_hbm, table_hbm, out_hbm, idx_v, rows_v, sem):
        wid = lax.axis_index("s") * _NC + lax.axis_index("c")
        pltpu.sync_copy(idx_hbm.at[wid], idx_v)
        base = wid * bpw

        def step(g, carry):
            pltpu.async_copy(table_hbm.at[idx_v.at[g]], rows_v, sem).wait()
            pltpu.sync_copy(rows_v, out_hbm.at[pl.ds(base + g * _CHUNK, _CHUNK)])
            return carry

        lax.fori_loop(0, steps, step, 0)

    return gather_kernel


def kernel(input_tokens, embedding):
    b, h = input_tokens.shape
    d = embedding.shape[1]
    B = b * h
    idx3 = input_tokens.reshape(_NW, B // (_NW * _CHUNK), _CHUNK).astype(jnp.int32)
    out = _make_gather(B, d)(idx3, embedding)
    return out.reshape(b, h, d)


# speedup vs baseline: 1.8725x; 1.1119x over previous
import functools

import jax
import jax.numpy as jnp
from jax import lax
from jax.experimental import pallas as pl
from jax.experimental.pallas import tpu as pltpu
from jax.experimental.pallas import tpu_sc as plsc

_NC = 2
_NS = 16
_NW = _NC * _NS
_CHUNK = 128


@functools.lru_cache(maxsize=None)
def _make_gather(B, D):
    assert B % (_NW * _CHUNK) == 0
    bpw = B // _NW
    steps = bpw // _CHUNK
    mesh = plsc.VectorSubcoreMesh(core_axis_name="c", subcore_axis_name="s")

    nbuf = 8

    @functools.partial(
        pl.kernel,
        out_type=jax.ShapeDtypeStruct((B, D), jnp.float32),
        mesh=mesh,
        compiler_params=pltpu.CompilerParams(use_tc_tiling_on_sc=False),
        scratch_types=[
            pltpu.VMEM((steps, _CHUNK), jnp.int32),
            pltpu.VMEM((nbuf, _CHUNK, D), jnp.float32),
            pltpu.SemaphoreType.DMA((nbuf,)),
        ],
    )
    def gather_kernel(idx_hbm, table_hbm, out_hbm, idx_v, rows_v, gsem):
        wid = lax.axis_index("s") * _NC + lax.axis_index("c")
        pltpu.sync_copy(idx_hbm.at[wid], idx_v)
        base = wid * bpw

        for b in range(nbuf):
            pltpu.async_copy(table_hbm.at[idx_v.at[b]], rows_v.at[b], gsem.at[b])

        def step(g, carry):
            slot = lax.rem(g, nbuf)
            pltpu.make_async_copy(
                table_hbm.at[idx_v.at[g]], rows_v.at[slot], gsem.at[slot]
            ).wait()
            pltpu.sync_copy(rows_v.at[slot], out_hbm.at[pl.ds(base + g * _CHUNK, _CHUNK)])

            @pl.when(g + nbuf < steps)
            def _():
                pltpu.async_copy(
                    table_hbm.at[idx_v.at[g + nbuf]], rows_v.at[slot], gsem.at[slot]
                )

            return carry

        lax.fori_loop(0, steps, step, 0)

    return gather_kernel


def kernel(input_tokens, embedding):
    b, h = input_tokens.shape
    d = embedding.shape[1]
    B = b * h
    idx3 = input_tokens.reshape(_NW, B // (_NW * _CHUNK), _CHUNK).astype(jnp.int32)
    out = _make_gather(B, d)(idx3, embedding)
    return out.reshape(b, h, d)


# speedup vs baseline: 1.8755x; 1.0016x over previous
import functools

import jax
import jax.numpy as jnp
from jax import lax
from jax.experimental import pallas as pl
from jax.experimental.pallas import tpu as pltpu
from jax.experimental.pallas import tpu_sc as plsc

_NC = 2
_NS = 16
_NW = _NC * _NS
_CHUNK = 128


@functools.lru_cache(maxsize=None)
def _make_gather(B, D):
    assert B % (_NW * _CHUNK) == 0
    bpw = B // _NW
    steps = bpw // _CHUNK
    mesh = plsc.VectorSubcoreMesh(core_axis_name="c", subcore_axis_name="s")

    nbuf = 8

    @functools.partial(
        pl.kernel,
        out_type=jax.ShapeDtypeStruct((B, D), jnp.float32),
        mesh=mesh,
        compiler_params=pltpu.CompilerParams(use_tc_tiling_on_sc=False),
        scratch_types=[
            pltpu.VMEM((steps, _CHUNK), jnp.int32),
            pltpu.VMEM((nbuf, _CHUNK, D), jnp.float32),
            pltpu.SemaphoreType.DMA((nbuf,)),
            pltpu.SemaphoreType.DMA((nbuf,)),
        ],
    )
    def gather_kernel(idx_hbm, table_hbm, out_hbm, idx_v, rows_v, gsem, ssem):
        wid = lax.axis_index("s") * _NC + lax.axis_index("c")
        pltpu.sync_copy(idx_hbm.at[wid], idx_v)
        base = wid * bpw
        lag = nbuf - 1

        def step(g, carry):
            @pl.when(g < steps)
            def _issue():
                slot = lax.rem(g, nbuf)

                @pl.when(g >= nbuf)
                def _():
                    pltpu.make_async_copy(
                        rows_v.at[slot],
                        out_hbm.at[pl.ds(base + (g - nbuf) * _CHUNK, _CHUNK)],
                        ssem.at[slot],
                    ).wait()

                pltpu.async_copy(table_hbm.at[idx_v.at[g]], rows_v.at[slot], gsem.at[slot])

            j = g - lag

            @pl.when(j >= 0)
            def _drain():
                slot = lax.rem(j, nbuf)
                pltpu.make_async_copy(
                    table_hbm.at[idx_v.at[slot]], rows_v.at[slot], gsem.at[slot]
                ).wait()
                pltpu.async_copy(
                    rows_v.at[slot], out_hbm.at[pl.ds(base + j * _CHUNK, _CHUNK)], ssem.at[slot]
                )

            return carry

        lax.fori_loop(0, steps + lag, step, 0)

        for b in range(nbuf):
            j = steps - nbuf + b
            slot = j % nbuf
            pltpu.make_async_copy(
                rows_v.at[slot], out_hbm.at[pl.ds(base + j * _CHUNK, _CHUNK)], ssem.at[slot]
            ).wait()

    return gather_kernel


def kernel(input_tokens, embedding):
    b, h = input_tokens.shape
    d = embedding.shape[1]
    B = b * h
    idx3 = input_tokens.reshape(_NW, B // (_NW * _CHUNK), _CHUNK).astype(jnp.int32)
    out = _make_gather(B, d)(idx3, embedding)
    return out.reshape(b, h, d)
